# Initial kernel scaffold; baseline (speedup 1.0000x reference)
import jax
import jax.numpy as jnp
from jax.experimental import pallas as pl


def kernel(x):
    raise NotImplementedError("write your pallas kernel here")


# speedup vs baseline: 1.9541x; 1.9541x over previous
import jax
import jax.numpy as jnp
from jax.experimental import pallas as pl
from jax.experimental.pallas import tpu as pltpu

S_BLK = 128
F_BLK = 1024


def _excl_cumsum_body(x_ref, o_ref, carry_ref):
    s = pl.program_id(2)

    @pl.when(s == 0)
    def _():
        carry_ref[...] = jnp.zeros_like(carry_ref)

    xb = x_ref[0]
    row = jax.lax.broadcasted_iota(jnp.int32, (S_BLK, S_BLK), 0)
    col = jax.lax.broadcasted_iota(jnp.int32, (S_BLK, S_BLK), 1)
    tri = (col < row).astype(xb.dtype)
    exc = jax.lax.dot(tri, xb, preferred_element_type=jnp.float32)
    o_ref[0] = exc + carry_ref[...]
    carry_ref[...] += exc[S_BLK - 1:S_BLK, :] + xb[S_BLK - 1:S_BLK, :]


def kernel(x):
    B, S, F = x.shape
    grid = (B, F // F_BLK, S // S_BLK)
    return pl.pallas_call(
        _excl_cumsum_body,
        grid=grid,
        in_specs=[pl.BlockSpec((1, S_BLK, F_BLK), lambda b, f, s: (b, s, f))],
        out_specs=pl.BlockSpec((1, S_BLK, F_BLK), lambda b, f, s: (b, s, f)),
        out_shape=jax.ShapeDtypeStruct(x.shape, x.dtype),
        scratch_shapes=[pltpu.VMEM((1, F_BLK), jnp.float32)],
        compiler_params=pltpu.CompilerParams(
            dimension_semantics=("arbitrary", "arbitrary", "arbitrary"),
        ),
    )(x)
